# baseline (speedup 1.0000x reference)
import functools
import math

import jax
import jax.numpy as jnp
from jax.experimental import pallas as pl
from jax.experimental.pallas import tpu as pltpu

_B = 2048
_E = 64
_K = 8
_ROWS = 72
_C = 128
_CAP_F = 1.25
_BT = 8


def _routing_kernel(rf_ref, wr_ref, tw_ref, ti_ref, imp_ref):
    logits = jax.lax.dot_general(
        rf_ref[...], wr_ref[...],
        dimension_numbers=(((1,), (1,)), ((), ())),
        preferred_element_type=jnp.float32)
    m = jnp.max(logits, axis=1, keepdims=True)
    ex = jnp.exp(logits - m)
    probs = ex / jnp.sum(ex, axis=1, keepdims=True)
    imp_ref[...] = jnp.sum(probs, axis=0, keepdims=True)
    iota = jax.lax.broadcasted_iota(jnp.int32, probs.shape, 1)
    work = probs
    vals, idxs = [], []
    for _ in range(_K):
        mv = jnp.max(work, axis=1, keepdims=True)
        ik = jnp.min(jnp.where(work == mv, iota, jnp.int32(2 ** 30)),
                     axis=1, keepdims=True)
        vals.append(mv)
        idxs.append(ik)
        work = jnp.where(iota == ik, jnp.float32(-1.0), work)
    tv = jnp.concatenate(vals, axis=1)
    ti = jnp.concatenate(idxs, axis=1)
    tw_ref[...] = tv / (jnp.sum(tv, axis=1, keepdims=True) + 1e-9)
    ti_ref[...] = ti


def _capacity_kernel(cap, wf_ref, ef_ref, imp_ref, kept_ref, aux_ref):
    w = jnp.broadcast_to(wf_ref[...], (_E, _B * _K))
    e = jnp.broadcast_to(ef_ref[...], (_E, _B * _K))
    row = jax.lax.broadcasted_iota(jnp.int32, (_E, _B * _K), 0)
    scores = jnp.where(e == row, w, jnp.float32(-1.0))
    lo = jnp.zeros((_E, 1), jnp.int32)
    hi = jnp.full((_E, 1), 0x7F800000, jnp.int32)
    for _ in range(31):
        mid = lo + ((hi - lo) >> 1)
        midf = jax.lax.bitcast_convert_type(mid, jnp.float32)
        cnt = jnp.sum((scores >= midf).astype(jnp.float32), axis=1,
                      keepdims=True)
        pred = cnt >= jnp.float32(cap)
        lo = jnp.where(pred, mid, lo)
        hi = jnp.where(pred, hi, mid)
    tf = jax.lax.bitcast_convert_type(lo, jnp.float32)
    keptd = (scores >= tf).astype(jnp.float32)
    load = jnp.sum(keptd, axis=1, keepdims=True)
    kept_ref[...] = jnp.sum(keptd, axis=0, keepdims=True)
    imp = imp_ref[...]
    imp_n = imp / (jnp.sum(imp, keepdims=True) + 1e-9)
    load_n = load / (jnp.sum(load, keepdims=True) + 1e-9)
    aux_ref[...] = jnp.float32(_E) * jax.lax.dot_general(
        imp_n, load_n, dimension_numbers=(((1,), (0,)), ((), ())),
        preferred_element_type=jnp.float32,
        precision=jax.lax.Precision.HIGHEST)


def _renorm_kernel(tw_ref, kept_ref, wf_ref):
    wk = tw_ref[...] * kept_ref[...]
    wf_ref[...] = wk / (jnp.sum(wk, axis=1, keepdims=True) + 1e-9)


def _moe_kernel(ti_ref, wf_ref, x_ref, we_ref, be_ref, y_ref):
    y_ref[...] = jnp.zeros_like(y_ref)
    for j in range(_BT):
        for k in range(_K):
            wjk = wf_ref[j, k]

            @pl.when(wjk > 0.0)
            def _(j=j, k=k, wjk=wjk):
                ei = ti_ref[j, k]
                h = jax.lax.dot_general(
                    x_ref[j], we_ref[ei],
                    dimension_numbers=(((1,), (0,)), ((), ())),
                    preferred_element_type=jnp.float32,
                    precision=jax.lax.Precision.HIGHEST)
                h = h + be_ref[pl.ds(ei, 1), :]
                act = h * jax.nn.sigmoid(h)
                y_ref[j] = y_ref[j] + wjk * act


def kernel(node_embedding, router_fea, Wr, We, be):
    B, N, L2, C = node_embedding.shape
    E = Wr.shape[0]
    cap = int(math.ceil(_CAP_F * (B / E)))

    tw, ti, imp = pl.pallas_call(
        _routing_kernel,
        out_shape=(
            jax.ShapeDtypeStruct((B, _K), jnp.float32),
            jax.ShapeDtypeStruct((B, _K), jnp.int32),
            jax.ShapeDtypeStruct((1, E), jnp.float32),
        ),
    )(router_fea, Wr)

    kept_flat, aux = pl.pallas_call(
        functools.partial(_capacity_kernel, cap),
        out_shape=(
            jax.ShapeDtypeStruct((1, B * _K), jnp.float32),
            jax.ShapeDtypeStruct((1, 1), jnp.float32),
        ),
    )(tw.reshape(1, B * _K), ti.reshape(1, B * _K), imp)

    wf = pl.pallas_call(
        _renorm_kernel,
        out_shape=jax.ShapeDtypeStruct((B, _K), jnp.float32),
    )(tw, kept_flat.reshape(B, _K))

    x3 = node_embedding.reshape(B, N * L2, C)
    y3 = pl.pallas_call(
        _moe_kernel,
        grid=(B // _BT,),
        in_specs=[
            pl.BlockSpec((_BT, _K), lambda i: (i, 0),
                         memory_space=pltpu.MemorySpace.SMEM),
            pl.BlockSpec((_BT, _K), lambda i: (i, 0),
                         memory_space=pltpu.MemorySpace.SMEM),
            pl.BlockSpec((_BT, N * L2, C), lambda i: (i, 0, 0)),
            pl.BlockSpec((E, C, C), lambda i: (0, 0, 0)),
            pl.BlockSpec((E, C), lambda i: (0, 0)),
        ],
        out_specs=pl.BlockSpec((_BT, N * L2, C), lambda i: (i, 0, 0)),
        out_shape=jax.ShapeDtypeStruct((B, N * L2, C), jnp.float32),
    )(ti, wf, x3, We, be)

    return y3.reshape(B, N, L2, C), aux.reshape(())


# speedup vs baseline: 1.4757x; 1.4757x over previous
import math

import jax
import jax.numpy as jnp
from jax import lax
from jax.experimental import pallas as pl
from jax.experimental.pallas import tpu as pltpu

_B = 2048
_E = 64
_K = 8
_ROWS = 72
_C = 128
_CAP_F = 1.25
_BT = 16


def _route_kernel(rf_ref, wr_ref, ce_ref, cw_ref, n_ref, aux_ref):
    logits = lax.dot_general(
        rf_ref[...], wr_ref[...],
        dimension_numbers=(((1,), (1,)), ((), ())),
        preferred_element_type=jnp.float32)
    m = jnp.max(logits, axis=1, keepdims=True)
    ex = jnp.exp(logits - m)
    probs = ex / jnp.sum(ex, axis=1, keepdims=True)

    iota = lax.broadcasted_iota(jnp.int32, (_B, _E), 1)
    work = probs
    vals, idxs = [], []
    for _ in range(_K):
        mv = jnp.max(work, axis=1, keepdims=True)
        ik = jnp.min(jnp.where(work == mv, iota, jnp.int32(2 ** 30)),
                     axis=1, keepdims=True)
        vals.append(mv)
        idxs.append(ik)
        work = jnp.where(iota == ik, jnp.float32(-1.0), work)

    denom = vals[0] + vals[1] + vals[2] + vals[3] + \
        vals[4] + vals[5] + vals[6] + vals[7]
    inv = 1.0 / (denom + 1e-9)
    mask8 = work < -0.5
    cand = jnp.where(mask8, probs * inv, jnp.float32(-1.0))

    cap_f = jnp.float32(math.ceil(_CAP_F * (_B / _E)))
    lo = jnp.zeros((1, _E), jnp.int32)
    hi = jnp.full((1, _E), 0x3F800001, jnp.int32)
    for _ in range(30):
        mid = lo + ((hi - lo) >> 1)
        midf = lax.bitcast_convert_type(mid, jnp.float32)
        cnt = jnp.sum((cand >= midf).astype(jnp.float32), axis=0,
                      keepdims=True)
        pred = cnt >= cap_f
        lo = jnp.where(pred, mid, lo)
        hi = jnp.where(pred, hi, mid)
    tf = lax.bitcast_convert_type(lo, jnp.float32)

    kept_dense = (cand >= tf).astype(jnp.float32)

    kept_k = []
    s2 = jnp.zeros((_B, 1), jnp.float32)
    for k in range(_K):
        wn_k = vals[k] * inv
        onehot = (iota == idxs[k]).astype(jnp.float32)
        tg = jnp.sum(onehot * tf, axis=1, keepdims=True)
        kk = wn_k >= tg
        kept_k.append(kk)
        s2 = s2 + jnp.where(kk, wn_k, 0.0)
    inv2 = 1.0 / (s2 + 1e-9)

    lane8 = lax.broadcasted_iota(jnp.int32, (_B, _K), 1)
    ce = jnp.zeros((_B, _K), jnp.int32)
    cw = jnp.zeros((_B, _K), jnp.float32)
    rank = jnp.zeros((_B, 1), jnp.int32)
    for k in range(_K):
        put = kept_k[k] & (lane8 == rank)
        ce = jnp.where(put, idxs[k], ce)
        cw = jnp.where(put, vals[k] * inv * inv2, cw)
        rank = rank + kept_k[k].astype(jnp.int32)
    ce_ref[...] = ce
    cw_ref[...] = cw
    n_ref[...] = rank

    imp = jnp.sum(probs, axis=0, keepdims=True)
    load = jnp.sum(kept_dense, axis=0, keepdims=True)
    imp_n = imp / (jnp.sum(imp, keepdims=True) + 1e-9)
    load_n = load / (jnp.sum(load, keepdims=True) + 1e-9)
    aux_ref[...] = jnp.sum(jnp.float32(_E) * imp_n * load_n,
                           keepdims=True)


def _moe_kernel(ce_ref, cw_ref, n_ref, x_ref, we_ref, be_ref, y_ref):
    for j in range(_BT):
        nj = n_ref[j, 0]

        def body(k, acc, j=j):
            e = ce_ref[j, k]
            w = cw_ref[j, k]
            h = lax.dot_general(
                x_ref[j], we_ref[e],
                dimension_numbers=(((1,), (0,)), ((), ())),
                preferred_element_type=jnp.float32)
            h = h + be_ref[pl.ds(e, 1), :]
            return acc + w * (h * jax.nn.sigmoid(h))

        y_ref[j] = lax.fori_loop(0, nj, body,
                                 jnp.zeros((_ROWS, _C), jnp.float32))


def kernel(node_embedding, router_fea, Wr, We, be):
    B, N, L2, C = node_embedding.shape
    E = Wr.shape[0]

    ce, cw, n, aux = pl.pallas_call(
        _route_kernel,
        out_shape=(
            jax.ShapeDtypeStruct((B, _K), jnp.int32),
            jax.ShapeDtypeStruct((B, _K), jnp.float32),
            jax.ShapeDtypeStruct((B, 1), jnp.int32),
            jax.ShapeDtypeStruct((1, 1), jnp.float32),
        ),
    )(router_fea, Wr)

    x3 = node_embedding.reshape(B, N * L2, C)
    y3 = pl.pallas_call(
        _moe_kernel,
        grid=(B // _BT,),
        in_specs=[
            pl.BlockSpec((_BT, _K), lambda i: (i, 0),
                         memory_space=pltpu.MemorySpace.SMEM),
            pl.BlockSpec((_BT, _K), lambda i: (i, 0),
                         memory_space=pltpu.MemorySpace.SMEM),
            pl.BlockSpec((_BT, 1), lambda i: (i, 0),
                         memory_space=pltpu.MemorySpace.SMEM),
            pl.BlockSpec((_BT, N * L2, C), lambda i: (i, 0, 0)),
            pl.BlockSpec((E, C, C), lambda i: (0, 0, 0)),
            pl.BlockSpec((E, C), lambda i: (0, 0)),
        ],
        out_specs=pl.BlockSpec((_BT, N * L2, C), lambda i: (i, 0, 0)),
        out_shape=jax.ShapeDtypeStruct((B, N * L2, C), jnp.float32),
    )(ce, cw, n, x3, We, be)

    return y3.reshape(B, N, L2, C), aux.reshape(())


# speedup vs baseline: 1.9771x; 1.3398x over previous
import math

import jax
import jax.numpy as jnp
from jax import lax
from jax.experimental import pallas as pl
from jax.experimental.pallas import tpu as pltpu

_B = 2048
_E = 64
_K = 8
_ROWS = 72
_C = 128
_CAP = int(math.ceil(1.25 * _B / _E))
_CHUNK = 8


def _route_kernel(rf_ref, wr_ref, tok_ref, wsel_ref, aux_ref):
    logits = lax.dot_general(
        rf_ref[...], wr_ref[...],
        dimension_numbers=(((1,), (1,)), ((), ())),
        preferred_element_type=jnp.float32)
    m = jnp.max(logits, axis=1, keepdims=True)
    ex = jnp.exp(logits - m)
    probs = ex / jnp.sum(ex, axis=1, keepdims=True)

    iota = lax.broadcasted_iota(jnp.int32, (_B, _E), 1)
    work = probs
    vals = []
    for _ in range(_K):
        mv = jnp.max(work, axis=1, keepdims=True)
        ik = jnp.min(jnp.where(work == mv, iota, jnp.int32(2 ** 30)),
                     axis=1, keepdims=True)
        vals.append(mv)
        work = jnp.where(iota == ik, jnp.float32(-1.0), work)

    denom = vals[0] + vals[1] + vals[2] + vals[3] + \
        vals[4] + vals[5] + vals[6] + vals[7]
    inv = 1.0 / (denom + 1e-9)
    mask8 = work < -0.5
    cand = jnp.where(mask8, probs * inv, jnp.float32(-1.0))

    lo = jnp.zeros((1, _E), jnp.int32)
    hi = jnp.full((1, _E), 0x3F800001, jnp.int32)
    for _ in range(30):
        mid = lo + ((hi - lo) >> 1)
        midf = lax.bitcast_convert_type(mid, jnp.float32)
        cnt = jnp.sum((cand >= midf).astype(jnp.float32), axis=0,
                      keepdims=True)
        pred = cnt >= jnp.float32(_CAP)
        lo = jnp.where(pred, mid, lo)
        hi = jnp.where(pred, hi, mid)
    tf = lax.bitcast_convert_type(lo, jnp.float32)

    kept = (cand >= tf).astype(jnp.float32)

    s2 = jnp.sum(kept * jnp.maximum(cand, 0.0), axis=1, keepdims=True)
    wf = kept * jnp.maximum(cand, 0.0) / (s2 + 1e-9)

    incl = kept
    sh = 1
    while sh < _B:
        incl = incl + jnp.concatenate(
            [jnp.zeros((sh, _E), jnp.float32), incl[:-sh]], axis=0)
        sh *= 2
    r = incl - kept

    tok_f = lax.broadcasted_iota(jnp.int32, (_B, _E), 0).astype(jnp.float32)
    tok_rows, w_rows = [], []
    for s in range(_CAP):
        m_s = kept * (r == jnp.float32(s)).astype(jnp.float32)
        tok_rows.append(jnp.sum(m_s * tok_f, axis=0, keepdims=True))
        w_rows.append(jnp.sum(m_s * wf, axis=0, keepdims=True))
    tok_ref[...] = jnp.concatenate(tok_rows, axis=0).astype(jnp.int32)
    wsel_ref[...] = jnp.concatenate(w_rows, axis=0)

    imp = jnp.sum(probs, axis=0, keepdims=True)
    load = jnp.sum(kept, axis=0, keepdims=True)
    imp_n = imp / (jnp.sum(imp, keepdims=True) + 1e-9)
    load_n = load / (jnp.sum(load, keepdims=True) + 1e-9)
    aux_ref[...] = jnp.sum(jnp.float32(_E) * imp_n * load_n,
                           keepdims=True)


def _zero_kernel(y_ref):
    y_ref[...] = jnp.zeros_like(y_ref)


def _moe_kernel(tok_ref, w_ref, xh_ref, we_ref, be_ref, yin_ref, yo_ref,
                xbuf, ybuf, semx, semyr, semyw):
    del yin_ref
    e = pl.program_id(0)
    cur = lax.rem(e, 2)
    nxt = 1 - cur

    def issue_x(expert, buf):
        for i in range(_CAP):
            t = tok_ref[i, expert]
            pltpu.make_async_copy(
                xh_ref.at[t], xbuf.at[buf, i], semx.at[buf]).start()

    def wait_x(buf):
        for i in range(_CAP):
            pltpu.make_async_copy(
                xh_ref.at[0], xbuf.at[buf, i], semx.at[buf]).wait()

    def drain_yw(buf):
        for i in range(_CAP):
            pltpu.make_async_copy(
                ybuf.at[buf, i], yo_ref.at[0], semyw.at[buf]).wait()

    @pl.when(e == 0)
    def _():
        issue_x(e, cur)

    @pl.when(e + 1 < _E)
    def _():
        issue_x(e + 1, nxt)

    @pl.when(e > 0)
    def _():
        drain_yw(nxt)

    for i in range(_CAP):
        t = tok_ref[i, e]
        pltpu.make_async_copy(
            yo_ref.at[t], ybuf.at[cur, i], semyr.at[cur]).start()

    wait_x(cur)

    for i in range(_CAP):
        pltpu.make_async_copy(
            yo_ref.at[0], ybuf.at[cur, i], semyr.at[cur]).wait()

    for c in range(_CAP // _CHUNK):
        xc = xbuf[cur, pl.ds(c * _CHUNK, _CHUNK)]
        xr = xc.reshape(_CHUNK * _ROWS, _C)
        h = lax.dot_general(
            xr, we_ref[0],
            dimension_numbers=(((1,), (0,)), ((), ())),
            preferred_element_type=jnp.float32)
        h = h + be_ref[pl.ds(e, 1), :]
        act = h * jax.nn.sigmoid(h)
        for i in range(_CHUNK):
            s = c * _CHUNK + i
            wv = w_ref[s, e]
            ybuf[cur, s] = ybuf[cur, s] + wv * act[i * _ROWS:(i + 1) * _ROWS]

    for i in range(_CAP):
        t = tok_ref[i, e]
        pltpu.make_async_copy(
            ybuf.at[cur, i], yo_ref.at[t], semyw.at[cur]).start()

    @pl.when(e == _E - 1)
    def _():
        drain_yw(cur)


def kernel(node_embedding, router_fea, Wr, We, be):
    B, N, L2, C = node_embedding.shape
    E = Wr.shape[0]

    tok, wsel, aux = pl.pallas_call(
        _route_kernel,
        out_shape=(
            jax.ShapeDtypeStruct((_CAP, E), jnp.int32),
            jax.ShapeDtypeStruct((_CAP, E), jnp.float32),
            jax.ShapeDtypeStruct((1, 1), jnp.float32),
        ),
    )(router_fea, Wr)

    y0 = pl.pallas_call(
        _zero_kernel,
        grid=(16,),
        out_specs=pl.BlockSpec((B // 16, N * L2, C), lambda i: (i, 0, 0)),
        out_shape=jax.ShapeDtypeStruct((B, N * L2, C), jnp.float32),
    )()

    x3 = node_embedding.reshape(B, N * L2, C)
    y3 = pl.pallas_call(
        _moe_kernel,
        grid=(E,),
        in_specs=[
            pl.BlockSpec((_CAP, E), lambda e: (0, 0),
                         memory_space=pltpu.MemorySpace.SMEM),
            pl.BlockSpec((_CAP, E), lambda e: (0, 0),
                         memory_space=pltpu.MemorySpace.SMEM),
            pl.BlockSpec(memory_space=pltpu.MemorySpace.HBM),
            pl.BlockSpec((1, C, C), lambda e: (e, 0, 0)),
            pl.BlockSpec((E, C), lambda e: (0, 0)),
            pl.BlockSpec(memory_space=pltpu.MemorySpace.HBM),
        ],
        out_specs=pl.BlockSpec(memory_space=pltpu.MemorySpace.HBM),
        out_shape=jax.ShapeDtypeStruct((B, N * L2, C), jnp.float32),
        scratch_shapes=[
            pltpu.VMEM((2, _CAP, _ROWS, _C), jnp.float32),
            pltpu.VMEM((2, _CAP, _ROWS, _C), jnp.float32),
            pltpu.SemaphoreType.DMA((2,)),
            pltpu.SemaphoreType.DMA((2,)),
            pltpu.SemaphoreType.DMA((2,)),
        ],
        input_output_aliases={5: 0},
    )(tok, wsel, x3, We, be, y0)

    return y3.reshape(B, N, L2, C), aux.reshape(())
